# Initial kernel scaffold; baseline (speedup 1.0000x reference)
import jax
import jax.numpy as jnp
from jax.experimental import pallas as pl


def kernel(features, labels, Proto):
    raise NotImplementedError("write your pallas kernel here")


# speedup vs baseline: 2.7900x; 2.7900x over previous
import functools

import jax
import jax.numpy as jnp
from jax import lax
from jax.experimental import pallas as pl
from jax.experimental.pallas import tpu as pltpu
from jax.experimental.pallas import tpu_sc as plsc

N = 524288
D = 256
C = 19
ACC_ROWS = 40
NW = 32
ROWS_PER_TILE = N // NW
CHUNK = 128
NPAIR = ROWS_PER_TILE // (2 * CHUNK)
LANES = 16
GRP = D // LANES

MOM = 0.9
W_NEW = 1.0 - MOM


def _sc_body(feat_hbm, lab_hbm, out_hbm,
             fbuf0, fbuf1, lbuf0, lbuf1, acc, cnt,
             fsem0, fsem1, lsem0, lsem1):
  wid = lax.axis_index("s") * 2 + lax.axis_index("c")
  base = wid * ROWS_PER_TILE

  zeros = jnp.zeros((LANES,), jnp.float32)
  def _zrow(i, _):
    for j in range(GRP):
      acc[i, pl.ds(j * LANES, LANES)] = zeros
    return 0
  lax.fori_loop(0, C + 1, _zrow, 0)
  def _zcnt(i, _):
    cnt[i] = 0.0
    return 0
  lax.fori_loop(0, C + 1, _zcnt, 0)

  def start(c, fbuf, lbuf, fsem, lsem):
    row0 = base + c * CHUNK
    pltpu.async_copy(feat_hbm.at[pl.ds(row0, CHUNK)], fbuf, fsem)
    pltpu.async_copy(lab_hbm.at[pl.ds(row0, CHUNK)], lbuf, lsem)

  def wait(c, fbuf, lbuf, fsem, lsem):
    row0 = base + c * CHUNK
    pltpu.make_async_copy(feat_hbm.at[pl.ds(row0, CHUNK)], fbuf, fsem).wait()
    pltpu.make_async_copy(lab_hbm.at[pl.ds(row0, CHUNK)], lbuf, lsem).wait()

  def process(fbuf, lbuf):
    def grp(g, _):
      lv = lbuf[pl.ds(g * LANES, LANES)]
      for k in range(LANES):
        lbl = lv[k]
        c0 = cnt[lbl]
        cnt[lbl] = c0 + 1.0
        r = g * LANES + k
        for j in range(GRP):
          v = fbuf[r, pl.ds(j * LANES, LANES)]
          plsc.addupdate(acc.at[lbl, pl.ds(j * LANES, LANES)], v)
      return 0
    lax.fori_loop(0, CHUNK // LANES, grp, 0)

  start(0, fbuf0, lbuf0, fsem0, lsem0)

  def pair(i, _):
    c0 = 2 * i
    start(c0 + 1, fbuf1, lbuf1, fsem1, lsem1)
    wait(c0, fbuf0, lbuf0, fsem0, lsem0)
    process(fbuf0, lbuf0)

    @pl.when(i < NPAIR - 1)
    def _():
      start(c0 + 2, fbuf0, lbuf0, fsem0, lsem0)

    wait(c0 + 1, fbuf1, lbuf1, fsem1, lsem1)
    process(fbuf1, lbuf1)
    return 0

  lax.fori_loop(0, NPAIR, pair, 0)

  for cls in range(C):
    vec = jnp.full((LANES,), cnt[cls], jnp.float32)
    for j in range(GRP):
      acc[C + 1 + cls, pl.ds(j * LANES, LANES)] = vec

  pltpu.sync_copy(acc, out_hbm.at[wid])


@functools.cache
def _sc_partials():
  return pl.kernel(
      _sc_body,
      out_type=jax.ShapeDtypeStruct((NW, ACC_ROWS, D), jnp.float32),
      mesh=plsc.VectorSubcoreMesh(core_axis_name="c", subcore_axis_name="s",
                                  num_cores=2, num_subcores=16),
      scratch_types=[
        pltpu.VMEM((CHUNK, D), jnp.float32),
        pltpu.VMEM((CHUNK, D), jnp.float32),
        pltpu.VMEM((CHUNK,), jnp.int32),
        pltpu.VMEM((CHUNK,), jnp.int32),
        pltpu.VMEM((ACC_ROWS, D), jnp.float32),
        pltpu.SMEM((C + 1,), jnp.float32),
        pltpu.SemaphoreType.DMA,
        pltpu.SemaphoreType.DMA,
        pltpu.SemaphoreType.DMA,
        pltpu.SemaphoreType.DMA,
      ],
  )


def _combine_body(p_ref, proto_ref, o_ref):
  p = p_ref[...]
  sums = jnp.sum(p[:, 0:C, :], axis=0)
  cnts = jnp.sum(p[:, C + 1:C + 1 + C, :], axis=0)
  mean = sums / jnp.maximum(cnts, 1.0)
  proto = proto_ref[...]
  o_ref[...] = jnp.where(cnts > 0.0, W_NEW * mean + MOM * proto, proto)


def kernel(features, labels, Proto):
  partials = _sc_partials()(features, labels)
  return pl.pallas_call(
      _combine_body,
      out_shape=jax.ShapeDtypeStruct((C, D), jnp.float32),
  )(partials, Proto)


# speedup vs baseline: 3.2960x; 1.1813x over previous
import functools

import jax
import jax.numpy as jnp
from jax import lax
from jax.experimental import pallas as pl
from jax.experimental.pallas import tpu as pltpu
from jax.experimental.pallas import tpu_sc as plsc

N = 524288
D = 256
C = 19
CR = 24
NW = 32
NS = 16
ROWS_PER_TILE = N // NW
CHUNK = 128
NPAIR = ROWS_PER_TILE // (2 * CHUNK)
LANES = 16
GRP = D // LANES

MOM = 0.9
W_NEW = 1.0 - MOM


def _sc_body(feat_hbm, lab_hbm, sums_hbm,
             fbuf0, fbuf1, lbv0, lbv1, acc,
             fsem0, fsem1, lsem0, lsem1):
  cid = lax.axis_index("c")
  sid = lax.axis_index("s")
  wid = sid * 2 + cid
  base = wid * ROWS_PER_TILE

  zeros = jnp.zeros((LANES,), jnp.float32)
  def _zrow(i, _):
    for j in range(GRP):
      acc[i, pl.ds(j * LANES, LANES)] = zeros
    return 0
  lax.fori_loop(0, CR, _zrow, 0)

  def start(c, fbuf, lbv, fsem, lsem):
    row0 = base + c * CHUNK
    pltpu.async_copy(feat_hbm.at[pl.ds(row0, CHUNK)], fbuf, fsem)
    pltpu.async_copy(lab_hbm.at[pl.ds(row0, CHUNK)], lbv, lsem)

  def wait(c, fbuf, lbv, fsem, lsem):
    row0 = base + c * CHUNK
    pltpu.make_async_copy(feat_hbm.at[pl.ds(row0, CHUNK)], fbuf, fsem).wait()
    pltpu.make_async_copy(lab_hbm.at[pl.ds(row0, CHUNK)], lbv, lsem).wait()

  def process(fbuf, lbuf):
    @plsc.parallel_loop(0, CHUNK // LANES)
    def _grp(g):
      lv = lbuf[pl.ds(g * LANES, LANES)]
      for k in range(LANES):
        lbl = lv[k]
        r = g * LANES + k
        for j in range(GRP):
          v = fbuf[r, pl.ds(j * LANES, LANES)]
          plsc.addupdate(acc.at[lbl, pl.ds(j * LANES, LANES)], v)

  start(0, fbuf0, lbv0, fsem0, lsem0)

  def pair(i, _):
    c0 = 2 * i
    start(c0 + 1, fbuf1, lbv1, fsem1, lsem1)
    wait(c0, fbuf0, lbv0, fsem0, lsem0)
    process(fbuf0, lbv0)

    @pl.when(i < NPAIR - 1)
    def _():
      start(c0 + 2, fbuf0, lbv0, fsem0, lsem0)

    wait(c0 + 1, fbuf1, lbv1, fsem1, lsem1)
    process(fbuf1, lbv1)
    return 0

  lax.fori_loop(0, NPAIR, pair, 0)

  pltpu.sync_copy(acc, sums_hbm.at[wid])


@functools.cache
def _sc_partials():
  return pl.kernel(
      _sc_body,
      out_type=jax.ShapeDtypeStruct((NW, CR, D), jnp.float32),
      mesh=plsc.VectorSubcoreMesh(core_axis_name="c", subcore_axis_name="s",
                                  num_cores=2, num_subcores=NS),
      scratch_types=[
        pltpu.VMEM((CHUNK, D), jnp.float32),
        pltpu.VMEM((CHUNK, D), jnp.float32),
        pltpu.VMEM((CHUNK,), jnp.int32),
        pltpu.VMEM((CHUNK,), jnp.int32),
        pltpu.VMEM((CR, D), jnp.float32),
        pltpu.SemaphoreType.DMA,
        pltpu.SemaphoreType.DMA,
        pltpu.SemaphoreType.DMA,
        pltpu.SemaphoreType.DMA,
      ],
  )


def _combine_body(sums_ref, lab_ref, proto_ref, o_ref):
  sums = jnp.sum(sums_ref[...], axis=0)[:C]
  labs = lab_ref[...]
  cnts = jnp.stack(
      [jnp.sum(jnp.where(labs == c, 1.0, 0.0)) for c in range(C)]
  )[:, None]
  mean = sums / jnp.maximum(cnts, 1.0)
  proto = proto_ref[...]
  o_ref[...] = jnp.where(cnts > 0.0, W_NEW * mean + MOM * proto, proto)


def kernel(features, labels, Proto):
  sums = _sc_partials()(features, labels)
  labs2d = labels.reshape(N // 128, 128)
  return pl.pallas_call(
      _combine_body,
      out_shape=jax.ShapeDtypeStruct((C, D), jnp.float32),
  )(sums, labs2d, Proto)


# speedup vs baseline: 3.3110x; 1.0046x over previous
import functools

import jax
import jax.numpy as jnp
from jax import lax
from jax.experimental import pallas as pl
from jax.experimental.pallas import tpu as pltpu
from jax.experimental.pallas import tpu_sc as plsc

N = 524288
D = 256
C = 19
CR = 24
NW = 32
NS = 16
ROWS_PER_TILE = N // NW
CHUNK = 128
NPAIR = ROWS_PER_TILE // (2 * CHUNK)
LANES = 16
GRP = D // LANES

MOM = 0.9
W_NEW = 1.0 - MOM


def _sc_body(feat_hbm, lab_hbm, sums_hbm,
             fbuf0, fbuf1, lbv0, lbv1, acc, accb,
             fsem0, fsem1, lsem0, lsem1):
  cid = lax.axis_index("c")
  sid = lax.axis_index("s")
  wid = sid * 2 + cid
  base = wid * ROWS_PER_TILE

  zeros = jnp.zeros((LANES,), jnp.float32)
  def _zrow(i, _):
    for j in range(GRP):
      acc[i, pl.ds(j * LANES, LANES)] = zeros
      accb[i, pl.ds(j * LANES, LANES)] = zeros
    return 0
  lax.fori_loop(0, CR, _zrow, 0)

  def start(c, fbuf, lbv, fsem, lsem):
    row0 = base + c * CHUNK
    pltpu.async_copy(feat_hbm.at[pl.ds(row0, CHUNK)], fbuf, fsem)
    pltpu.async_copy(lab_hbm.at[pl.ds(row0, CHUNK)], lbv, lsem)

  def wait(c, fbuf, lbv, fsem, lsem):
    row0 = base + c * CHUNK
    pltpu.make_async_copy(feat_hbm.at[pl.ds(row0, CHUNK)], fbuf, fsem).wait()
    pltpu.make_async_copy(lab_hbm.at[pl.ds(row0, CHUNK)], lbv, lsem).wait()

  def process(fbuf, lbuf):
    @plsc.parallel_loop(0, CHUNK // LANES)
    def _grp(g):
      lv = lbuf[pl.ds(g * LANES, LANES)]
      for k in range(LANES):
        lbl = lv[k]
        r = g * LANES + k
        dst = acc if k % 2 == 0 else accb
        for j in range(GRP):
          v = fbuf[r, pl.ds(j * LANES, LANES)]
          plsc.addupdate(dst.at[lbl, pl.ds(j * LANES, LANES)], v)

  start(0, fbuf0, lbv0, fsem0, lsem0)

  def pair(i, _):
    c0 = 2 * i
    start(c0 + 1, fbuf1, lbv1, fsem1, lsem1)
    wait(c0, fbuf0, lbv0, fsem0, lsem0)
    process(fbuf0, lbv0)

    @pl.when(i < NPAIR - 1)
    def _():
      start(c0 + 2, fbuf0, lbv0, fsem0, lsem0)

    wait(c0 + 1, fbuf1, lbv1, fsem1, lsem1)
    process(fbuf1, lbv1)
    return 0

  lax.fori_loop(0, NPAIR, pair, 0)

  def _merge(i, _):
    for j in range(GRP):
      acc[i, pl.ds(j * LANES, LANES)] = (
          acc[i, pl.ds(j * LANES, LANES)] + accb[i, pl.ds(j * LANES, LANES)])
    return 0
  lax.fori_loop(0, CR, _merge, 0)

  pltpu.sync_copy(acc, sums_hbm.at[wid])


@functools.cache
def _sc_partials():
  return pl.kernel(
      _sc_body,
      out_type=jax.ShapeDtypeStruct((NW, CR, D), jnp.float32),
      mesh=plsc.VectorSubcoreMesh(core_axis_name="c", subcore_axis_name="s",
                                  num_cores=2, num_subcores=NS),
      scratch_types=[
        pltpu.VMEM((CHUNK, D), jnp.float32),
        pltpu.VMEM((CHUNK, D), jnp.float32),
        pltpu.VMEM((CHUNK,), jnp.int32),
        pltpu.VMEM((CHUNK,), jnp.int32),
        pltpu.VMEM((CR, D), jnp.float32),
        pltpu.VMEM((CR, D), jnp.float32),
        pltpu.SemaphoreType.DMA,
        pltpu.SemaphoreType.DMA,
        pltpu.SemaphoreType.DMA,
        pltpu.SemaphoreType.DMA,
      ],
  )


def _combine_body(sums_ref, lab_ref, proto_ref, o_ref):
  sums = jnp.sum(sums_ref[...], axis=0)[:C]
  labs = lab_ref[...]
  cnts = jnp.stack(
      [jnp.sum(jnp.where(labs == c, 1.0, 0.0)) for c in range(C)]
  )[:, None]
  mean = sums / jnp.maximum(cnts, 1.0)
  proto = proto_ref[...]
  o_ref[...] = jnp.where(cnts > 0.0, W_NEW * mean + MOM * proto, proto)


def kernel(features, labels, Proto):
  sums = _sc_partials()(features, labels)
  labs2d = labels.reshape(N // 128, 128)
  return pl.pallas_call(
      _combine_body,
      out_shape=jax.ShapeDtypeStruct((C, D), jnp.float32),
  )(sums, labs2d, Proto)


# speedup vs baseline: 6.2329x; 1.8825x over previous
import functools

import jax
import jax.numpy as jnp
from jax import lax
from jax.experimental import pallas as pl
from jax.experimental.pallas import tpu as pltpu
from jax.experimental.pallas import tpu_sc as plsc

N = 524288
D = 256
C = 19
CR = 24
NW = 32
NS = 16
ROWS_PER_TILE = N // NW
CHUNK = 128
NPAIR = ROWS_PER_TILE // (2 * CHUNK)
LANES = 16
GRP = D // LANES

MOM = 0.9
W_NEW = 1.0 - MOM


def _sc_body(feat_hbm, lab_hbm, sums_hbm,
             fbuf0, fbuf1, lbv0, lbv1, acc, accb,
             fsem0, fsem1, lsem0, lsem1):
  cid = lax.axis_index("c")
  sid = lax.axis_index("s")
  wid = sid * 2 + cid
  base = wid * ROWS_PER_TILE

  zeros = jnp.zeros((LANES,), jnp.float32)
  def _zrow(i, _):
    for j in range(GRP):
      acc[i, pl.ds(j * LANES, LANES)] = zeros
      accb[i, pl.ds(j * LANES, LANES)] = zeros
    return 0
  lax.fori_loop(0, CR, _zrow, 0)

  def start(c, fbuf, lbv, fsem, lsem):
    row0 = base + c * CHUNK
    pltpu.async_copy(feat_hbm.at[pl.ds(row0, CHUNK)], fbuf, fsem)
    pltpu.async_copy(lab_hbm.at[pl.ds(row0, CHUNK)], lbv, lsem)

  def wait(c, fbuf, lbv, fsem, lsem):
    row0 = base + c * CHUNK
    pltpu.make_async_copy(feat_hbm.at[pl.ds(row0, CHUNK)], fbuf, fsem).wait()
    pltpu.make_async_copy(lab_hbm.at[pl.ds(row0, CHUNK)], lbv, lsem).wait()

  def process(fbuf, lbuf):
    @plsc.parallel_loop(0, CHUNK // LANES)
    def _grp(g):
      lv = lbuf[pl.ds(g * LANES, LANES)]
      for k in range(LANES):
        lbl = lv[k]
        r = g * LANES + k
        dst = acc if k % 2 == 0 else accb
        vs = [fbuf[r, pl.ds(j * LANES, LANES)] for j in range(GRP)]
        for j in range(GRP):
          plsc.addupdate(dst.at[lbl, pl.ds(j * LANES, LANES)], vs[j])

  start(0, fbuf0, lbv0, fsem0, lsem0)

  def pair(i, _):
    c0 = 2 * i
    start(c0 + 1, fbuf1, lbv1, fsem1, lsem1)
    wait(c0, fbuf0, lbv0, fsem0, lsem0)
    process(fbuf0, lbv0)

    @pl.when(i < NPAIR - 1)
    def _():
      start(c0 + 2, fbuf0, lbv0, fsem0, lsem0)

    wait(c0 + 1, fbuf1, lbv1, fsem1, lsem1)
    process(fbuf1, lbv1)
    return 0

  lax.fori_loop(0, NPAIR, pair, 0)

  def _merge(i, _):
    for j in range(GRP):
      acc[i, pl.ds(j * LANES, LANES)] = (
          acc[i, pl.ds(j * LANES, LANES)] + accb[i, pl.ds(j * LANES, LANES)])
    return 0
  lax.fori_loop(0, CR, _merge, 0)

  pltpu.sync_copy(acc, sums_hbm.at[wid])


@functools.cache
def _sc_partials():
  return pl.kernel(
      _sc_body,
      out_type=jax.ShapeDtypeStruct((NW, CR, D), jnp.float32),
      mesh=plsc.VectorSubcoreMesh(core_axis_name="c", subcore_axis_name="s",
                                  num_cores=2, num_subcores=NS),
      scratch_types=[
        pltpu.VMEM((CHUNK, D), jnp.float32),
        pltpu.VMEM((CHUNK, D), jnp.float32),
        pltpu.VMEM((CHUNK,), jnp.int32),
        pltpu.VMEM((CHUNK,), jnp.int32),
        pltpu.VMEM((CR, D), jnp.float32),
        pltpu.VMEM((CR, D), jnp.float32),
        pltpu.SemaphoreType.DMA,
        pltpu.SemaphoreType.DMA,
        pltpu.SemaphoreType.DMA,
        pltpu.SemaphoreType.DMA,
      ],
  )


def _combine_body(sums_ref, lab_ref, proto_ref, o_ref):
  sums = jnp.sum(sums_ref[...], axis=0)[:C]
  labs = lab_ref[...]
  cnts = jnp.stack(
      [jnp.sum(jnp.where(labs == c, 1.0, 0.0)) for c in range(C)]
  )[:, None]
  mean = sums / jnp.maximum(cnts, 1.0)
  proto = proto_ref[...]
  o_ref[...] = jnp.where(cnts > 0.0, W_NEW * mean + MOM * proto, proto)


def kernel(features, labels, Proto):
  sums = _sc_partials()(features, labels)
  labs2d = labels.reshape(N // 128, 128)
  return pl.pallas_call(
      _combine_body,
      out_shape=jax.ShapeDtypeStruct((C, D), jnp.float32),
  )(sums, labs2d, Proto)
